# baseline (speedup 1.0000x reference)
import functools

import jax
import jax.numpy as jnp
from jax import lax
from jax.experimental import pallas as pl
from jax.experimental.pallas import tpu as pltpu
from jax.experimental.pallas import tpu_sc as plsc

HEADS = 8
HEAD_DIM = 16
LANES = 16
NEG_SLOPE = 0.2

NUM_CORES = 2
NUM_SUBCORES = 16
CHUNK = 32
CPB = 8
NSLOTS = 4
ROW_W = HEADS * HEAD_DIM + LANES


def _leaky(x):
    return jnp.maximum(x, x * NEG_SLOPE)


def _tc_pre_body(x_ref, w_ref, asrc_ref, adst_ref, g_ref, dt_ref, c_ref):
    x = x_ref[:]
    xp = jnp.dot(x, w_ref[:], preferred_element_type=jnp.float32)
    ps = xp * asrc_ref[:]
    pd = xp * adst_ref[:]
    col = lax.broadcasted_iota(jnp.int32, (HEADS * HEAD_DIM, HEADS), 0)
    hh = lax.broadcasted_iota(jnp.int32, (HEADS * HEAD_DIM, HEADS), 1)
    ones_blk = jnp.where((col // HEAD_DIM) == hh, 1.0, 0.0)
    ones_rev = jnp.where((col // HEAD_DIM) == (HEADS - 1 - hh), 1.0, 0.0)
    a_src = jnp.dot(ps, ones_blk, preferred_element_type=jnp.float32)
    a_dst = jnp.dot(pd, ones_blk, preferred_element_type=jnp.float32)
    a_src_r = jnp.dot(ps, ones_rev, preferred_element_type=jnp.float32)
    a_dst_r = jnp.dot(pd, ones_rev, preferred_element_type=jnp.float32)
    amax = jnp.max(a_src, axis=0, keepdims=True)
    amax_r = jnp.max(a_src_r, axis=0, keepdims=True)
    c = _leaky(amax + a_dst)
    c_rev = _leaky(amax_r + a_dst_r)
    g_ref[:] = jnp.concatenate([a_src, jnp.zeros_like(a_src), xp], axis=1)
    dt_ref[:] = jnp.concatenate([a_dst, c_rev], axis=1)
    c_ref[:] = c


def _sc_edge_body(n_nodes, e_per_sub, nchunks,
                  g_hbm, dt_hbm, src_hbm, dst_hbm, z_hbm, p_hbm,
                  sidx_a, sidx_b, didx_a, didx_b, *bufs):
    gvs = bufs[0:NSLOTS]
    dbs = bufs[NSLOTS:2 * NSLOTS]
    mbs = bufs[2 * NSLOTS:3 * NSLOTS]
    acc = bufs[3 * NSLOTS]
    semg = bufs[3 * NSLOTS + 1:3 * NSLOTS + 1 + NSLOTS]
    semd = bufs[3 * NSLOTS + 1 + NSLOTS:3 * NSLOTS + 1 + 2 * NSLOTS]
    sems = bufs[3 * NSLOTS + 1 + 2 * NSLOTS:3 * NSLOTS + 1 + 3 * NSLOTS]
    cid = lax.axis_index("c")
    sid = lax.axis_index("s")

    stripe = (n_nodes // (NUM_SUBCORES * 8)) * 8
    r_zero = sid * stripe
    pltpu.sync_copy(z_hbm.at[pl.ds(r_zero, stripe)],
                    acc.at[pl.ds(r_zero, stripe)])
    rem = n_nodes - stripe * NUM_SUBCORES
    if rem:
        @pl.when(sid == NUM_SUBCORES - 1)
        def _():
            pltpu.sync_copy(z_hbm.at[pl.ds(stripe * NUM_SUBCORES, rem)],
                            acc.at[pl.ds(stripe * NUM_SUBCORES, rem)])
    plsc.subcore_barrier()

    a_lo = cid * (e_per_sub * NUM_SUBCORES) + sid * e_per_sub
    r0 = a_lo // CHUNK
    a64 = a_lo - r0 * CHUNK
    lane = lax.iota(jnp.int32, LANES)
    lo_mask = lane < HEADS

    idx_blks = ((sidx_a, didx_a), (sidx_b, didx_b))

    def copy_idx_block(first_chunk, blk):
        sblk, dblk = blk
        pltpu.sync_copy(src_hbm.at[pl.ds(r0 + first_chunk, CPB)], sblk)
        pltpu.sync_copy(dst_hbm.at[pl.ds(r0 + first_chunk, CPB)], dblk)

    def issue_gather(blk, jr, s):
        sblk, dblk = blk
        pltpu.async_copy(g_hbm.at[sblk.at[jr]], gvs[s], semg[s])
        pltpu.async_copy(dt_hbm.at[dblk.at[jr]], dbs[s], semd[s])

    def wait_gather(s):
        pltpu.make_async_copy(g_hbm.at[pl.ds(0, CHUNK)], gvs[s],
                              semg[s]).wait()
        pltpu.make_async_copy(dt_hbm.at[pl.ds(0, CHUNK)], dbs[s],
                              semd[s]).wait()

    def wait_scatter(s):
        pltpu.make_async_copy(mbs[s], acc.at[pl.ds(0, CHUNK)],
                              sems[s]).wait()

    def compute_chunk(j, blk, jr, s):
        gv, db, mb = gvs[s], dbs[s], mbs[s]
        _, dblk = blk
        lo = a64 - j * CHUNK
        hi = lo + e_per_sub

        def edge_body(k, carry):
            s16 = gv[k, pl.ds(0, LANES)]
            d16 = db[k, :]
            v = s16 + d16
            t = _leaky(v) - lax.rev(v, (0,))
            w = jnp.where(lo_mask, jnp.exp(t), 0.0)
            w = w * ((k >= lo) & (k < hi)).astype(jnp.float32)
            mb[k, pl.ds(0, LANES)] = w
            for h in range(HEADS):
                xv = gv[k, pl.ds(LANES + h * LANES, LANES)]
                mb[k, pl.ds(LANES + h * LANES, LANES)] = xv * w[h]
            return carry

        lax.fori_loop(0, CHUNK, edge_body, 0, unroll=2)
        pltpu.async_copy(mb, acc.at[dblk.at[jr]], sems[s], add=True)

    copy_idx_block(0, idx_blks[0])
    for s in range(NSLOTS):
        issue_gather(idx_blks[0], s, s)

    nsuper = nchunks // (2 * CPB)

    def super_body(i, carry):
        for t in range(2 * CPB):
            j = i * (2 * CPB) + t
            s = t % NSLOTS
            blk = idx_blks[(t // CPB) % 2]
            wait_gather(s)

            @pl.when(j >= NSLOTS)
            def _():
                wait_scatter(s)

            compute_chunk(j, blk, t % CPB, s)

            tn = t + NSLOTS
            if tn == CPB:
                copy_idx_block(i * (2 * CPB) + CPB, idx_blks[1])
            if tn < 2 * CPB:
                issue_gather(idx_blks[(tn // CPB) % 2], tn % CPB, tn % NSLOTS)
            else:
                @pl.when(i < nsuper - 1)
                def _():
                    if tn == 2 * CPB:
                        copy_idx_block((i + 1) * (2 * CPB), idx_blks[0])
                    issue_gather(idx_blks[0], tn % CPB, tn % NSLOTS)
        return carry

    lax.fori_loop(0, nsuper, super_body, 0)
    for s in range(NSLOTS):
        wait_scatter(s)

    plsc.subcore_barrier()
    pltpu.sync_copy(acc.at[pl.ds(r_zero, stripe)],
                    p_hbm.at[cid, pl.ds(r_zero, stripe)])
    if rem:
        @pl.when(sid == NUM_SUBCORES - 1)
        def _():
            pltpu.sync_copy(acc.at[pl.ds(stripe * NUM_SUBCORES, rem)],
                            p_hbm.at[cid, pl.ds(stripe * NUM_SUBCORES, rem)])


def _tc_stats_body(p_ref, g_ref, dt_ref, c_ref, bias_ref, outr_ref, sums_ref):
    p0 = p_ref[0]
    p1 = p_ref[1]
    g = g_ref[:]
    a_src = g[:, 0:HEADS]
    xp = g[:, LANES:LANES + HEADS * HEAD_DIM]
    a_dst = dt_ref[:][:, 0:HEADS]
    wself = jnp.exp(_leaky(a_src + a_dst) - c_ref[:])
    den = p0[:, 0:HEADS] + p1[:, 0:HEADS] + wself

    col = lax.broadcasted_iota(jnp.int32, (HEADS, HEADS * HEAD_DIM), 1)
    hh = lax.broadcasted_iota(jnp.int32, (HEADS, HEAD_DIM * HEADS), 0)
    expand = jnp.where((col // HEAD_DIM) == hh, 1.0, 0.0)

    msg = (p0[:, LANES:] + p1[:, LANES:]
           + jnp.dot(wself, expand, preferred_element_type=jnp.float32) * xp)
    out = msg / (jnp.dot(den, expand, preferred_element_type=jnp.float32)
                 + 1e-16)
    out = out + bias_ref[:]
    outr_ref[:] = out
    sums_ref[0] = jnp.concatenate(
        [jnp.sum(out, axis=0, keepdims=True),
         jnp.sum(out * out, axis=0, keepdims=True)], axis=0)


def _tc_norm_body(n_rows, outr_ref, sums_ref, gamma_ref, beta_ref, out_ref):
    out = outr_ref[:]
    s = jnp.sum(sums_ref[:, 0, :], axis=0, keepdims=True)
    s2 = jnp.sum(sums_ref[:, 1, :], axis=0, keepdims=True)
    mean = s / n_rows
    var = s2 / n_rows - mean * mean
    out = (out - mean) * lax.rsqrt(var + 1e-5) * gamma_ref[:] + beta_ref[:]
    out_ref[:] = jnp.maximum(out, 0.0)


def kernel(x_gnn, edge_index, W, att_src, att_dst, bias, gamma, beta):
    n, in_ch = x_gnn.shape
    e = edge_index.shape[1]
    out_ch = W.shape[1]
    src = edge_index[0].astype(jnp.int32)
    dst = edge_index[1].astype(jnp.int32)

    g, dt, c = pl.pallas_call(
        _tc_pre_body,
        out_shape=[
            jax.ShapeDtypeStruct((n, ROW_W), jnp.float32),
            jax.ShapeDtypeStruct((n, LANES), jnp.float32),
            jax.ShapeDtypeStruct((n, HEADS), jnp.float32),
        ],
    )(x_gnn, W, att_src.reshape(1, out_ch), att_dst.reshape(1, out_ch))

    e_per_sub = e // (NUM_CORES * NUM_SUBCORES)
    nchunks = -(-e_per_sub // CHUNK) + 1
    nchunks = -(-nchunks // (2 * CPB)) * (2 * CPB)
    nrows = -(-e // CHUNK) + 2 * CPB
    pad = nrows * CHUNK - e
    src2 = jnp.concatenate([src, jnp.zeros((pad,), jnp.int32)]).reshape(
        nrows, CHUNK)
    dst2 = jnp.concatenate([dst, jnp.zeros((pad,), jnp.int32)]).reshape(
        nrows, CHUNK)
    zeros = jnp.zeros((n, ROW_W), jnp.float32)

    sc_call = pl.kernel(
        functools.partial(_sc_edge_body, n, e_per_sub, nchunks),
        out_type=jax.ShapeDtypeStruct((NUM_CORES, n, ROW_W), jnp.float32),
        mesh=plsc.VectorSubcoreMesh(core_axis_name="c", subcore_axis_name="s"),
        compiler_params=pltpu.CompilerParams(
            needs_layout_passes=False, use_tc_tiling_on_sc=False),
        scratch_types=(
            [pltpu.VMEM((CPB, CHUNK), jnp.int32)] * 4
            + [pltpu.VMEM((CHUNK, ROW_W), jnp.float32)] * NSLOTS
            + [pltpu.VMEM((CHUNK, LANES), jnp.float32)] * NSLOTS
            + [pltpu.VMEM((CHUNK, ROW_W), jnp.float32)] * NSLOTS
            + [pltpu.VMEM_SHARED((n, ROW_W), jnp.float32)]
            + [pltpu.SemaphoreType.DMA] * (3 * NSLOTS)
        ),
    )
    p = sc_call(g, dt, src2, dst2, zeros)

    blk = 1000
    nblk = n // blk
    outr, sums = pl.pallas_call(
        _tc_stats_body,
        grid=(nblk,),
        in_specs=[
            pl.BlockSpec((NUM_CORES, blk, ROW_W), lambda i: (0, i, 0)),
            pl.BlockSpec((blk, ROW_W), lambda i: (i, 0)),
            pl.BlockSpec((blk, LANES), lambda i: (i, 0)),
            pl.BlockSpec((blk, HEADS), lambda i: (i, 0)),
            pl.BlockSpec((1, out_ch), lambda i: (0, 0)),
        ],
        out_specs=[
            pl.BlockSpec((blk, out_ch), lambda i: (i, 0)),
            pl.BlockSpec((1, 2, out_ch), lambda i: (i, 0, 0)),
        ],
        out_shape=[
            jax.ShapeDtypeStruct((n, out_ch), jnp.float32),
            jax.ShapeDtypeStruct((nblk, 2, out_ch), jnp.float32),
        ],
    )(p, g, dt, c, bias.reshape(1, out_ch))

    out = pl.pallas_call(
        functools.partial(_tc_norm_body, float(n)),
        grid=(nblk,),
        in_specs=[
            pl.BlockSpec((blk, out_ch), lambda i: (i, 0)),
            pl.BlockSpec((nblk, 2, out_ch), lambda i: (0, 0, 0)),
            pl.BlockSpec((1, out_ch), lambda i: (0, 0)),
            pl.BlockSpec((1, out_ch), lambda i: (0, 0)),
        ],
        out_specs=pl.BlockSpec((blk, out_ch), lambda i: (i, 0)),
        out_shape=jax.ShapeDtypeStruct((n, out_ch), jnp.float32),
    )(outr, sums, gamma.reshape(1, out_ch), beta.reshape(1, out_ch))
    return out


# speedup vs baseline: 1.6438x; 1.6438x over previous
import functools

import jax
import jax.numpy as jnp
from jax import lax
from jax.experimental import pallas as pl
from jax.experimental.pallas import tpu as pltpu
from jax.experimental.pallas import tpu_sc as plsc

HEADS = 8
HEAD_DIM = 16
LANES = 16
NEG_SLOPE = 0.2

NUM_CORES = 2
NUM_SUBCORES = 16
CHUNK = 32
CPB = 8
NSLOTS = 4
ROW_W = HEADS * HEAD_DIM + LANES


def _leaky(x):
    return jnp.maximum(x, x * NEG_SLOPE)


def _tc_pre_body(x_ref, w_ref, asrc_ref, adst_ref, g_ref, dt_ref, c_ref):
    x = x_ref[:]
    xp = jnp.dot(x, w_ref[:], preferred_element_type=jnp.float32)
    ps = xp * asrc_ref[:]
    pd = xp * adst_ref[:]
    col = lax.broadcasted_iota(jnp.int32, (HEADS * HEAD_DIM, HEADS), 0)
    hh = lax.broadcasted_iota(jnp.int32, (HEADS * HEAD_DIM, HEADS), 1)
    ones_blk = jnp.where((col // HEAD_DIM) == hh, 1.0, 0.0)
    ones_rev = jnp.where((col // HEAD_DIM) == (HEADS - 1 - hh), 1.0, 0.0)
    a_src = jnp.dot(ps, ones_blk, preferred_element_type=jnp.float32)
    a_dst = jnp.dot(pd, ones_blk, preferred_element_type=jnp.float32)
    a_src_r = jnp.dot(ps, ones_rev, preferred_element_type=jnp.float32)
    a_dst_r = jnp.dot(pd, ones_rev, preferred_element_type=jnp.float32)
    amax = jnp.max(a_src, axis=0, keepdims=True)
    amax_r = jnp.max(a_src_r, axis=0, keepdims=True)
    c = _leaky(amax + a_dst)
    c_rev = _leaky(amax_r + a_dst_r)
    g_ref[:] = jnp.concatenate([a_src, jnp.zeros_like(a_src), xp], axis=1)
    dt_ref[:] = jnp.concatenate([a_dst, c_rev], axis=1)
    c_ref[:] = c


def _sc_edge_body(n_nodes, e_per_sub, nchunks,
                  g_hbm, dt_hbm, src_hbm, dst_hbm, z_hbm, p_hbm,
                  sidx_a, sidx_b, didx_a, didx_b, *bufs):
    gvs = bufs[0:NSLOTS]
    dbs = bufs[NSLOTS:2 * NSLOTS]
    mbs = bufs[2 * NSLOTS:3 * NSLOTS]
    acc = bufs[3 * NSLOTS]
    semg = bufs[3 * NSLOTS + 1:3 * NSLOTS + 1 + NSLOTS]
    semd = bufs[3 * NSLOTS + 1 + NSLOTS:3 * NSLOTS + 1 + 2 * NSLOTS]
    sems = bufs[3 * NSLOTS + 1 + 2 * NSLOTS:3 * NSLOTS + 1 + 3 * NSLOTS]
    cid = lax.axis_index("c")
    sid = lax.axis_index("s")

    stripe = (n_nodes // (NUM_SUBCORES * 8)) * 8
    r_zero = sid * stripe
    pltpu.sync_copy(z_hbm.at[pl.ds(r_zero, stripe)],
                    acc.at[pl.ds(r_zero, stripe)])
    rem = n_nodes - stripe * NUM_SUBCORES
    if rem:
        @pl.when(sid == NUM_SUBCORES - 1)
        def _():
            pltpu.sync_copy(z_hbm.at[pl.ds(stripe * NUM_SUBCORES, rem)],
                            acc.at[pl.ds(stripe * NUM_SUBCORES, rem)])
    plsc.subcore_barrier()

    a_lo = cid * (e_per_sub * NUM_SUBCORES) + sid * e_per_sub
    r0 = a_lo // CHUNK
    a64 = a_lo - r0 * CHUNK
    lane = lax.iota(jnp.int32, LANES)
    lo_mask = lane < HEADS

    idx_blks = ((sidx_a, didx_a), (sidx_b, didx_b))

    def copy_idx_block(first_chunk, blk):
        sblk, dblk = blk
        pltpu.sync_copy(src_hbm.at[pl.ds(r0 + first_chunk, CPB)], sblk)
        pltpu.sync_copy(dst_hbm.at[pl.ds(r0 + first_chunk, CPB)], dblk)

    def issue_gather(blk, jr, s):
        sblk, dblk = blk
        pltpu.async_copy(g_hbm.at[sblk.at[jr]], gvs[s], semg[s])
        pltpu.async_copy(dt_hbm.at[dblk.at[jr]], dbs[s], semd[s])

    def wait_gather(s):
        pltpu.make_async_copy(g_hbm.at[pl.ds(0, CHUNK)], gvs[s],
                              semg[s]).wait()
        pltpu.make_async_copy(dt_hbm.at[pl.ds(0, CHUNK)], dbs[s],
                              semd[s]).wait()

    def wait_scatter(s):
        pltpu.make_async_copy(mbs[s], acc.at[pl.ds(0, CHUNK)],
                              sems[s]).wait()

    def compute_chunk(j, blk, jr, s):
        gv, db, mb = gvs[s], dbs[s], mbs[s]
        _, dblk = blk
        lo = a64 - j * CHUNK
        hi = lo + e_per_sub

        def edge_body(k, carry):
            s16 = gv[k, pl.ds(0, LANES)]
            d16 = db[k, :]
            v = s16 + d16
            t = _leaky(v) - lax.rev(v, (0,))
            w = jnp.where(lo_mask, jnp.exp(t), 0.0)
            w = w * ((k >= lo) & (k < hi)).astype(jnp.float32)
            mb[k, pl.ds(0, LANES)] = w
            xvs = [gv[k, pl.ds(LANES + h * LANES, LANES)]
                   for h in range(HEADS)]
            ws = [w[h] for h in range(HEADS)]
            for h in range(HEADS):
                mb[k, pl.ds(LANES + h * LANES, LANES)] = xvs[h] * ws[h]
            return carry

        lax.fori_loop(0, CHUNK, edge_body, 0, unroll=2)
        pltpu.async_copy(mb, acc.at[dblk.at[jr]], sems[s], add=True)

    copy_idx_block(0, idx_blks[0])
    for s in range(NSLOTS):
        issue_gather(idx_blks[0], s, s)

    nsuper = nchunks // (2 * CPB)

    def super_body(i, carry):
        for t in range(2 * CPB):
            j = i * (2 * CPB) + t
            s = t % NSLOTS
            blk = idx_blks[(t // CPB) % 2]
            wait_gather(s)

            @pl.when(j >= NSLOTS)
            def _():
                wait_scatter(s)

            compute_chunk(j, blk, t % CPB, s)

            tn = t + NSLOTS
            if tn == CPB:
                copy_idx_block(i * (2 * CPB) + CPB, idx_blks[1])
            if tn < 2 * CPB:
                issue_gather(idx_blks[(tn // CPB) % 2], tn % CPB, tn % NSLOTS)
            else:
                @pl.when(i < nsuper - 1)
                def _():
                    if tn == 2 * CPB:
                        copy_idx_block((i + 1) * (2 * CPB), idx_blks[0])
                    issue_gather(idx_blks[0], tn % CPB, tn % NSLOTS)
        return carry

    lax.fori_loop(0, nsuper, super_body, 0)
    for s in range(NSLOTS):
        wait_scatter(s)

    plsc.subcore_barrier()
    pltpu.sync_copy(acc.at[pl.ds(r_zero, stripe)],
                    p_hbm.at[cid, pl.ds(r_zero, stripe)])
    if rem:
        @pl.when(sid == NUM_SUBCORES - 1)
        def _():
            pltpu.sync_copy(acc.at[pl.ds(stripe * NUM_SUBCORES, rem)],
                            p_hbm.at[cid, pl.ds(stripe * NUM_SUBCORES, rem)])


def _tc_stats_body(p_ref, g_ref, dt_ref, c_ref, bias_ref, outr_ref, sums_ref):
    p0 = p_ref[0]
    p1 = p_ref[1]
    g = g_ref[:]
    a_src = g[:, 0:HEADS]
    xp = g[:, LANES:LANES + HEADS * HEAD_DIM]
    a_dst = dt_ref[:][:, 0:HEADS]
    wself = jnp.exp(_leaky(a_src + a_dst) - c_ref[:])
    den = p0[:, 0:HEADS] + p1[:, 0:HEADS] + wself

    col = lax.broadcasted_iota(jnp.int32, (HEADS, HEADS * HEAD_DIM), 1)
    hh = lax.broadcasted_iota(jnp.int32, (HEADS, HEAD_DIM * HEADS), 0)
    expand = jnp.where((col // HEAD_DIM) == hh, 1.0, 0.0)

    msg = (p0[:, LANES:] + p1[:, LANES:]
           + jnp.dot(wself, expand, preferred_element_type=jnp.float32) * xp)
    out = msg / (jnp.dot(den, expand, preferred_element_type=jnp.float32)
                 + 1e-16)
    out = out + bias_ref[:]
    outr_ref[:] = out
    sums_ref[0] = jnp.concatenate(
        [jnp.sum(out, axis=0, keepdims=True),
         jnp.sum(out * out, axis=0, keepdims=True)], axis=0)


def _tc_norm_body(n_rows, outr_ref, sums_ref, gamma_ref, beta_ref, out_ref):
    out = outr_ref[:]
    s = jnp.sum(sums_ref[:, 0, :], axis=0, keepdims=True)
    s2 = jnp.sum(sums_ref[:, 1, :], axis=0, keepdims=True)
    mean = s / n_rows
    var = s2 / n_rows - mean * mean
    out = (out - mean) * lax.rsqrt(var + 1e-5) * gamma_ref[:] + beta_ref[:]
    out_ref[:] = jnp.maximum(out, 0.0)


def kernel(x_gnn, edge_index, W, att_src, att_dst, bias, gamma, beta):
    n, in_ch = x_gnn.shape
    e = edge_index.shape[1]
    out_ch = W.shape[1]
    src = edge_index[0].astype(jnp.int32)
    dst = edge_index[1].astype(jnp.int32)

    g, dt, c = pl.pallas_call(
        _tc_pre_body,
        out_shape=[
            jax.ShapeDtypeStruct((n, ROW_W), jnp.float32),
            jax.ShapeDtypeStruct((n, LANES), jnp.float32),
            jax.ShapeDtypeStruct((n, HEADS), jnp.float32),
        ],
    )(x_gnn, W, att_src.reshape(1, out_ch), att_dst.reshape(1, out_ch))

    e_per_sub = e // (NUM_CORES * NUM_SUBCORES)
    nchunks = -(-e_per_sub // CHUNK) + 1
    nchunks = -(-nchunks // (2 * CPB)) * (2 * CPB)
    nrows = -(-e // CHUNK) + 2 * CPB
    pad = nrows * CHUNK - e
    src2 = jnp.concatenate([src, jnp.zeros((pad,), jnp.int32)]).reshape(
        nrows, CHUNK)
    dst2 = jnp.concatenate([dst, jnp.zeros((pad,), jnp.int32)]).reshape(
        nrows, CHUNK)
    zeros = jnp.zeros((n, ROW_W), jnp.float32)

    sc_call = pl.kernel(
        functools.partial(_sc_edge_body, n, e_per_sub, nchunks),
        out_type=jax.ShapeDtypeStruct((NUM_CORES, n, ROW_W), jnp.float32),
        mesh=plsc.VectorSubcoreMesh(core_axis_name="c", subcore_axis_name="s"),
        compiler_params=pltpu.CompilerParams(
            needs_layout_passes=False, use_tc_tiling_on_sc=False),
        scratch_types=(
            [pltpu.VMEM((CPB, CHUNK), jnp.int32)] * 4
            + [pltpu.VMEM((CHUNK, ROW_W), jnp.float32)] * NSLOTS
            + [pltpu.VMEM((CHUNK, LANES), jnp.float32)] * NSLOTS
            + [pltpu.VMEM((CHUNK, ROW_W), jnp.float32)] * NSLOTS
            + [pltpu.VMEM_SHARED((n, ROW_W), jnp.float32)]
            + [pltpu.SemaphoreType.DMA] * (3 * NSLOTS)
        ),
    )
    p = sc_call(g, dt, src2, dst2, zeros)

    blk = 1000
    nblk = n // blk
    outr, sums = pl.pallas_call(
        _tc_stats_body,
        grid=(nblk,),
        in_specs=[
            pl.BlockSpec((NUM_CORES, blk, ROW_W), lambda i: (0, i, 0)),
            pl.BlockSpec((blk, ROW_W), lambda i: (i, 0)),
            pl.BlockSpec((blk, LANES), lambda i: (i, 0)),
            pl.BlockSpec((blk, HEADS), lambda i: (i, 0)),
            pl.BlockSpec((1, out_ch), lambda i: (0, 0)),
        ],
        out_specs=[
            pl.BlockSpec((blk, out_ch), lambda i: (i, 0)),
            pl.BlockSpec((1, 2, out_ch), lambda i: (i, 0, 0)),
        ],
        out_shape=[
            jax.ShapeDtypeStruct((n, out_ch), jnp.float32),
            jax.ShapeDtypeStruct((nblk, 2, out_ch), jnp.float32),
        ],
    )(p, g, dt, c, bias.reshape(1, out_ch))

    out = pl.pallas_call(
        functools.partial(_tc_norm_body, float(n)),
        grid=(nblk,),
        in_specs=[
            pl.BlockSpec((blk, out_ch), lambda i: (i, 0)),
            pl.BlockSpec((nblk, 2, out_ch), lambda i: (0, 0, 0)),
            pl.BlockSpec((1, out_ch), lambda i: (0, 0)),
            pl.BlockSpec((1, out_ch), lambda i: (0, 0)),
        ],
        out_specs=pl.BlockSpec((blk, out_ch), lambda i: (i, 0)),
        out_shape=jax.ShapeDtypeStruct((n, out_ch), jnp.float32),
    )(outr, sums, gamma.reshape(1, out_ch), beta.reshape(1, out_ch))
    return out


# speedup vs baseline: 1.6674x; 1.0144x over previous
import functools

import jax
import jax.numpy as jnp
from jax import lax
from jax.experimental import pallas as pl
from jax.experimental.pallas import tpu as pltpu
from jax.experimental.pallas import tpu_sc as plsc

HEADS = 8
HEAD_DIM = 16
LANES = 16
NEG_SLOPE = 0.2

NUM_CORES = 2
NUM_SUBCORES = 16
CHUNK = 32
CPB = 8
NSLOTS = 4
ROW_W = HEADS * HEAD_DIM + LANES


def _leaky(x):
    return jnp.maximum(x, x * NEG_SLOPE)


def _tc_pre_body(x_ref, w_ref, asrc_ref, adst_ref, g_ref, dt_ref, c_ref):
    x = x_ref[:]
    xp = jnp.dot(x, w_ref[:], preferred_element_type=jnp.float32)
    ps = xp * asrc_ref[:]
    pd = xp * adst_ref[:]
    col = lax.broadcasted_iota(jnp.int32, (HEADS * HEAD_DIM, HEADS), 0)
    hh = lax.broadcasted_iota(jnp.int32, (HEADS * HEAD_DIM, HEADS), 1)
    ones_blk = jnp.where((col // HEAD_DIM) == hh, 1.0, 0.0)
    ones_rev = jnp.where((col // HEAD_DIM) == (HEADS - 1 - hh), 1.0, 0.0)
    a_src = jnp.dot(ps, ones_blk, preferred_element_type=jnp.float32)
    a_dst = jnp.dot(pd, ones_blk, preferred_element_type=jnp.float32)
    a_src_r = jnp.dot(ps, ones_rev, preferred_element_type=jnp.float32)
    a_dst_r = jnp.dot(pd, ones_rev, preferred_element_type=jnp.float32)
    amax = jnp.max(a_src, axis=0, keepdims=True)
    amax_r = jnp.max(a_src_r, axis=0, keepdims=True)
    c = _leaky(amax + a_dst)
    c_rev = _leaky(amax_r + a_dst_r)
    g_ref[:] = jnp.concatenate([a_src, jnp.zeros_like(a_src), xp], axis=1)
    dt_ref[:] = jnp.concatenate([a_dst, c_rev], axis=1)
    c_ref[:] = c


def _sc_edge_body(n_nodes, e_per_sub, nchunks,
                  g_hbm, dt_hbm, src_hbm, dst_hbm, z_hbm, p_hbm,
                  sidx_a, sidx_b, didx_a, didx_b, *bufs):
    gvs = bufs[0:NSLOTS]
    dbs = bufs[NSLOTS:2 * NSLOTS]
    mbs = bufs[2 * NSLOTS:3 * NSLOTS]
    acc = bufs[3 * NSLOTS]
    semg = bufs[3 * NSLOTS + 1:3 * NSLOTS + 1 + NSLOTS]
    semd = bufs[3 * NSLOTS + 1 + NSLOTS:3 * NSLOTS + 1 + 2 * NSLOTS]
    sems = bufs[3 * NSLOTS + 1 + 2 * NSLOTS:3 * NSLOTS + 1 + 3 * NSLOTS]
    cid = lax.axis_index("c")
    sid = lax.axis_index("s")

    stripe = (n_nodes // (NUM_SUBCORES * 8)) * 8
    r_zero = sid * stripe
    pltpu.sync_copy(z_hbm.at[pl.ds(r_zero, stripe)],
                    acc.at[pl.ds(r_zero, stripe)])
    rem = n_nodes - stripe * NUM_SUBCORES
    if rem:
        @pl.when(sid == NUM_SUBCORES - 1)
        def _():
            pltpu.sync_copy(z_hbm.at[pl.ds(stripe * NUM_SUBCORES, rem)],
                            acc.at[pl.ds(stripe * NUM_SUBCORES, rem)])
    plsc.subcore_barrier()

    a_lo = cid * (e_per_sub * NUM_SUBCORES) + sid * e_per_sub
    r0 = a_lo // CHUNK
    a64 = a_lo - r0 * CHUNK
    lane = lax.iota(jnp.int32, LANES)
    lo_mask = lane < HEADS

    idx_blks = ((sidx_a, didx_a), (sidx_b, didx_b))

    def copy_idx_block(first_chunk, blk):
        sblk, dblk = blk
        pltpu.sync_copy(src_hbm.at[pl.ds(r0 + first_chunk, CPB)], sblk)
        pltpu.sync_copy(dst_hbm.at[pl.ds(r0 + first_chunk, CPB)], dblk)

    def issue_gather(blk, jr, s):
        sblk, dblk = blk
        pltpu.async_copy(g_hbm.at[sblk.at[jr]], gvs[s], semg[s])
        pltpu.async_copy(dt_hbm.at[dblk.at[jr]], dbs[s], semd[s])

    def wait_gather(s):
        pltpu.make_async_copy(g_hbm.at[pl.ds(0, CHUNK)], gvs[s],
                              semg[s]).wait()
        pltpu.make_async_copy(dt_hbm.at[pl.ds(0, CHUNK)], dbs[s],
                              semd[s]).wait()

    def wait_scatter(s):
        pltpu.make_async_copy(mbs[s], acc.at[pl.ds(0, CHUNK)],
                              sems[s]).wait()

    def compute_chunk(j, blk, jr, s):
        gv, db, mb = gvs[s], dbs[s], mbs[s]
        _, dblk = blk
        lo = a64 - j * CHUNK
        hi = lo + e_per_sub

        def edge_body(k, carry):
            s16 = gv[k, pl.ds(0, LANES)]
            d16 = db[k, :]
            v = s16 + d16
            t = _leaky(v) - lax.rev(v, (0,))
            w = jnp.where(lo_mask, jnp.exp(t), 0.0)
            w = w * ((k >= lo) & (k < hi)).astype(jnp.float32)
            mb[k, pl.ds(0, LANES)] = w
            xvs = [gv[k, pl.ds(LANES + h * LANES, LANES)]
                   for h in range(HEADS)]
            ws = [w[h] for h in range(HEADS)]
            for h in range(HEADS):
                mb[k, pl.ds(LANES + h * LANES, LANES)] = xvs[h] * ws[h]
            return carry

        lax.fori_loop(0, CHUNK, edge_body, 0, unroll=4)
        pltpu.async_copy(mb, acc.at[dblk.at[jr]], sems[s], add=True)

    copy_idx_block(0, idx_blks[0])
    for s in range(NSLOTS):
        issue_gather(idx_blks[0], s, s)

    nsuper = nchunks // (2 * CPB)

    def super_body(i, carry):
        for t in range(2 * CPB):
            j = i * (2 * CPB) + t
            s = t % NSLOTS
            blk = idx_blks[(t // CPB) % 2]
            wait_gather(s)

            @pl.when(j >= NSLOTS)
            def _():
                wait_scatter(s)

            compute_chunk(j, blk, t % CPB, s)

            tn = t + NSLOTS
            if tn == CPB:
                copy_idx_block(i * (2 * CPB) + CPB, idx_blks[1])
            if tn < 2 * CPB:
                issue_gather(idx_blks[(tn // CPB) % 2], tn % CPB, tn % NSLOTS)
            else:
                @pl.when(i < nsuper - 1)
                def _():
                    if tn == 2 * CPB:
                        copy_idx_block((i + 1) * (2 * CPB), idx_blks[0])
                    issue_gather(idx_blks[0], tn % CPB, tn % NSLOTS)
        return carry

    lax.fori_loop(0, nsuper, super_body, 0)
    for s in range(NSLOTS):
        wait_scatter(s)

    plsc.subcore_barrier()
    pltpu.sync_copy(acc.at[pl.ds(r_zero, stripe)],
                    p_hbm.at[cid, pl.ds(r_zero, stripe)])
    if rem:
        @pl.when(sid == NUM_SUBCORES - 1)
        def _():
            pltpu.sync_copy(acc.at[pl.ds(stripe * NUM_SUBCORES, rem)],
                            p_hbm.at[cid, pl.ds(stripe * NUM_SUBCORES, rem)])


def _tc_stats_body(p_ref, g_ref, dt_ref, c_ref, bias_ref, outr_ref, sums_ref):
    p0 = p_ref[0]
    p1 = p_ref[1]
    g = g_ref[:]
    a_src = g[:, 0:HEADS]
    xp = g[:, LANES:LANES + HEADS * HEAD_DIM]
    a_dst = dt_ref[:][:, 0:HEADS]
    wself = jnp.exp(_leaky(a_src + a_dst) - c_ref[:])
    den = p0[:, 0:HEADS] + p1[:, 0:HEADS] + wself

    col = lax.broadcasted_iota(jnp.int32, (HEADS, HEADS * HEAD_DIM), 1)
    hh = lax.broadcasted_iota(jnp.int32, (HEADS, HEAD_DIM * HEADS), 0)
    expand = jnp.where((col // HEAD_DIM) == hh, 1.0, 0.0)

    msg = (p0[:, LANES:] + p1[:, LANES:]
           + jnp.dot(wself, expand, preferred_element_type=jnp.float32) * xp)
    out = msg / (jnp.dot(den, expand, preferred_element_type=jnp.float32)
                 + 1e-16)
    out = out + bias_ref[:]
    outr_ref[:] = out
    sums_ref[0] = jnp.concatenate(
        [jnp.sum(out, axis=0, keepdims=True),
         jnp.sum(out * out, axis=0, keepdims=True)], axis=0)


def _tc_norm_body(n_rows, outr_ref, sums_ref, gamma_ref, beta_ref, out_ref):
    out = outr_ref[:]
    s = jnp.sum(sums_ref[:, 0, :], axis=0, keepdims=True)
    s2 = jnp.sum(sums_ref[:, 1, :], axis=0, keepdims=True)
    mean = s / n_rows
    var = s2 / n_rows - mean * mean
    out = (out - mean) * lax.rsqrt(var + 1e-5) * gamma_ref[:] + beta_ref[:]
    out_ref[:] = jnp.maximum(out, 0.0)


def kernel(x_gnn, edge_index, W, att_src, att_dst, bias, gamma, beta):
    n, in_ch = x_gnn.shape
    e = edge_index.shape[1]
    out_ch = W.shape[1]
    src = edge_index[0].astype(jnp.int32)
    dst = edge_index[1].astype(jnp.int32)

    g, dt, c = pl.pallas_call(
        _tc_pre_body,
        out_shape=[
            jax.ShapeDtypeStruct((n, ROW_W), jnp.float32),
            jax.ShapeDtypeStruct((n, LANES), jnp.float32),
            jax.ShapeDtypeStruct((n, HEADS), jnp.float32),
        ],
    )(x_gnn, W, att_src.reshape(1, out_ch), att_dst.reshape(1, out_ch))

    e_per_sub = e // (NUM_CORES * NUM_SUBCORES)
    nchunks = -(-e_per_sub // CHUNK) + 1
    nchunks = -(-nchunks // (2 * CPB)) * (2 * CPB)
    nrows = -(-e // CHUNK) + 2 * CPB
    pad = nrows * CHUNK - e
    src2 = jnp.concatenate([src, jnp.zeros((pad,), jnp.int32)]).reshape(
        nrows, CHUNK)
    dst2 = jnp.concatenate([dst, jnp.zeros((pad,), jnp.int32)]).reshape(
        nrows, CHUNK)
    zeros = jnp.zeros((n, ROW_W), jnp.float32)

    sc_call = pl.kernel(
        functools.partial(_sc_edge_body, n, e_per_sub, nchunks),
        out_type=jax.ShapeDtypeStruct((NUM_CORES, n, ROW_W), jnp.float32),
        mesh=plsc.VectorSubcoreMesh(core_axis_name="c", subcore_axis_name="s"),
        compiler_params=pltpu.CompilerParams(
            needs_layout_passes=False, use_tc_tiling_on_sc=False),
        scratch_types=(
            [pltpu.VMEM((CPB, CHUNK), jnp.int32)] * 4
            + [pltpu.VMEM((CHUNK, ROW_W), jnp.float32)] * NSLOTS
            + [pltpu.VMEM((CHUNK, LANES), jnp.float32)] * NSLOTS
            + [pltpu.VMEM((CHUNK, ROW_W), jnp.float32)] * NSLOTS
            + [pltpu.VMEM_SHARED((n, ROW_W), jnp.float32)]
            + [pltpu.SemaphoreType.DMA] * (3 * NSLOTS)
        ),
    )
    p = sc_call(g, dt, src2, dst2, zeros)

    blk = 1000
    nblk = n // blk
    outr, sums = pl.pallas_call(
        _tc_stats_body,
        grid=(nblk,),
        in_specs=[
            pl.BlockSpec((NUM_CORES, blk, ROW_W), lambda i: (0, i, 0)),
            pl.BlockSpec((blk, ROW_W), lambda i: (i, 0)),
            pl.BlockSpec((blk, LANES), lambda i: (i, 0)),
            pl.BlockSpec((blk, HEADS), lambda i: (i, 0)),
            pl.BlockSpec((1, out_ch), lambda i: (0, 0)),
        ],
        out_specs=[
            pl.BlockSpec((blk, out_ch), lambda i: (i, 0)),
            pl.BlockSpec((1, 2, out_ch), lambda i: (i, 0, 0)),
        ],
        out_shape=[
            jax.ShapeDtypeStruct((n, out_ch), jnp.float32),
            jax.ShapeDtypeStruct((nblk, 2, out_ch), jnp.float32),
        ],
    )(p, g, dt, c, bias.reshape(1, out_ch))

    out = pl.pallas_call(
        functools.partial(_tc_norm_body, float(n)),
        grid=(nblk,),
        in_specs=[
            pl.BlockSpec((blk, out_ch), lambda i: (i, 0)),
            pl.BlockSpec((nblk, 2, out_ch), lambda i: (0, 0, 0)),
            pl.BlockSpec((1, out_ch), lambda i: (0, 0)),
            pl.BlockSpec((1, out_ch), lambda i: (0, 0)),
        ],
        out_specs=pl.BlockSpec((blk, out_ch), lambda i: (i, 0)),
        out_shape=jax.ShapeDtypeStruct((n, out_ch), jnp.float32),
    )(outr, sums, gamma.reshape(1, out_ch), beta.reshape(1, out_ch))
    return out


# speedup vs baseline: 1.7497x; 1.0493x over previous
import functools

import jax
import jax.numpy as jnp
from jax import lax
from jax.experimental import pallas as pl
from jax.experimental.pallas import tpu as pltpu
from jax.experimental.pallas import tpu_sc as plsc

HEADS = 8
HEAD_DIM = 16
LANES = 16
NEG_SLOPE = 0.2

NUM_CORES = 2
NUM_SUBCORES = 16
CHUNK = 64
CPB = 8
NSLOTS = 2
ROW_W = HEADS * HEAD_DIM + LANES


def _leaky(x):
    return jnp.maximum(x, x * NEG_SLOPE)


def _tc_pre_body(x_ref, w_ref, asrc_ref, adst_ref, g_ref, dt_ref, c_ref):
    x = x_ref[:]
    xp = jnp.dot(x, w_ref[:], preferred_element_type=jnp.float32)
    ps = xp * asrc_ref[:]
    pd = xp * adst_ref[:]
    col = lax.broadcasted_iota(jnp.int32, (HEADS * HEAD_DIM, HEADS), 0)
    hh = lax.broadcasted_iota(jnp.int32, (HEADS * HEAD_DIM, HEADS), 1)
    ones_blk = jnp.where((col // HEAD_DIM) == hh, 1.0, 0.0)
    ones_rev = jnp.where((col // HEAD_DIM) == (HEADS - 1 - hh), 1.0, 0.0)
    a_src = jnp.dot(ps, ones_blk, preferred_element_type=jnp.float32)
    a_dst = jnp.dot(pd, ones_blk, preferred_element_type=jnp.float32)
    a_src_r = jnp.dot(ps, ones_rev, preferred_element_type=jnp.float32)
    a_dst_r = jnp.dot(pd, ones_rev, preferred_element_type=jnp.float32)
    amax = jnp.max(a_src, axis=0, keepdims=True)
    amax_r = jnp.max(a_src_r, axis=0, keepdims=True)
    c = _leaky(amax + a_dst)
    c_rev = _leaky(amax_r + a_dst_r)
    g_ref[:] = jnp.concatenate([a_src, jnp.zeros_like(a_src), xp], axis=1)
    dt_ref[:] = jnp.concatenate([a_dst, c_rev], axis=1)
    c_ref[:] = c


def _sc_edge_body(n_nodes, e_per_sub, nchunks,
                  g_hbm, dt_hbm, src_hbm, dst_hbm, z_hbm, p_hbm,
                  sidx_a, sidx_b, didx_a, didx_b, *bufs):
    gvs = bufs[0:NSLOTS]
    dbs = bufs[NSLOTS:2 * NSLOTS]
    mbs = bufs[2 * NSLOTS:3 * NSLOTS]
    acc = bufs[3 * NSLOTS]
    semg = bufs[3 * NSLOTS + 1:3 * NSLOTS + 1 + NSLOTS]
    semd = bufs[3 * NSLOTS + 1 + NSLOTS:3 * NSLOTS + 1 + 2 * NSLOTS]
    sems = bufs[3 * NSLOTS + 1 + 2 * NSLOTS:3 * NSLOTS + 1 + 3 * NSLOTS]
    cid = lax.axis_index("c")
    sid = lax.axis_index("s")

    stripe = (n_nodes // (NUM_SUBCORES * 8)) * 8
    r_zero = sid * stripe
    pltpu.sync_copy(z_hbm.at[pl.ds(r_zero, stripe)],
                    acc.at[pl.ds(r_zero, stripe)])
    rem = n_nodes - stripe * NUM_SUBCORES
    if rem:
        @pl.when(sid == NUM_SUBCORES - 1)
        def _():
            pltpu.sync_copy(z_hbm.at[pl.ds(stripe * NUM_SUBCORES, rem)],
                            acc.at[pl.ds(stripe * NUM_SUBCORES, rem)])
    plsc.subcore_barrier()

    a_lo = cid * (e_per_sub * NUM_SUBCORES) + sid * e_per_sub
    r0 = a_lo // CHUNK
    a64 = a_lo - r0 * CHUNK
    lane = lax.iota(jnp.int32, LANES)
    lo_mask = lane < HEADS

    idx_blks = ((sidx_a, didx_a), (sidx_b, didx_b))

    def copy_idx_block(first_chunk, blk):
        sblk, dblk = blk
        pltpu.sync_copy(src_hbm.at[pl.ds(r0 + first_chunk, CPB)], sblk)
        pltpu.sync_copy(dst_hbm.at[pl.ds(r0 + first_chunk, CPB)], dblk)

    def issue_gather(blk, jr, s):
        sblk, dblk = blk
        pltpu.async_copy(g_hbm.at[sblk.at[jr]], gvs[s], semg[s])
        pltpu.async_copy(dt_hbm.at[dblk.at[jr]], dbs[s], semd[s])

    def wait_gather(s):
        pltpu.make_async_copy(g_hbm.at[pl.ds(0, CHUNK)], gvs[s],
                              semg[s]).wait()
        pltpu.make_async_copy(dt_hbm.at[pl.ds(0, CHUNK)], dbs[s],
                              semd[s]).wait()

    def wait_scatter(s):
        pltpu.make_async_copy(mbs[s], acc.at[pl.ds(0, CHUNK)],
                              sems[s]).wait()

    def compute_chunk(j, blk, jr, s):
        gv, db, mb = gvs[s], dbs[s], mbs[s]
        _, dblk = blk
        lo = a64 - j * CHUNK
        hi = lo + e_per_sub

        def edge_body(k, carry):
            s16 = gv[k, pl.ds(0, LANES)]
            d16 = db[k, :]
            v = s16 + d16
            t = _leaky(v) - lax.rev(v, (0,))
            w = jnp.where(lo_mask, jnp.exp(t), 0.0)
            w = w * ((k >= lo) & (k < hi)).astype(jnp.float32)
            mb[k, pl.ds(0, LANES)] = w
            xvs = [gv[k, pl.ds(LANES + h * LANES, LANES)]
                   for h in range(HEADS)]
            ws = [w[h] for h in range(HEADS)]
            for h in range(HEADS):
                mb[k, pl.ds(LANES + h * LANES, LANES)] = xvs[h] * ws[h]
            return carry

        lax.fori_loop(0, CHUNK, edge_body, 0, unroll=4)
        pltpu.async_copy(mb, acc.at[dblk.at[jr]], sems[s], add=True)

    copy_idx_block(0, idx_blks[0])
    for s in range(NSLOTS):
        issue_gather(idx_blks[0], s, s)

    nsuper = nchunks // (2 * CPB)

    def super_body(i, carry):
        for t in range(2 * CPB):
            j = i * (2 * CPB) + t
            s = t % NSLOTS
            blk = idx_blks[(t // CPB) % 2]
            wait_gather(s)

            @pl.when(j >= NSLOTS)
            def _():
                wait_scatter(s)

            compute_chunk(j, blk, t % CPB, s)

            tn = t + NSLOTS
            if tn == CPB:
                copy_idx_block(i * (2 * CPB) + CPB, idx_blks[1])
            if tn < 2 * CPB:
                issue_gather(idx_blks[(tn // CPB) % 2], tn % CPB, tn % NSLOTS)
            else:
                @pl.when(i < nsuper - 1)
                def _():
                    if tn == 2 * CPB:
                        copy_idx_block((i + 1) * (2 * CPB), idx_blks[0])
                    issue_gather(idx_blks[0], tn % CPB, tn % NSLOTS)
        return carry

    lax.fori_loop(0, nsuper, super_body, 0)
    for s in range(NSLOTS):
        wait_scatter(s)

    plsc.subcore_barrier()
    pltpu.sync_copy(acc.at[pl.ds(r_zero, stripe)],
                    p_hbm.at[cid, pl.ds(r_zero, stripe)])
    if rem:
        @pl.when(sid == NUM_SUBCORES - 1)
        def _():
            pltpu.sync_copy(acc.at[pl.ds(stripe * NUM_SUBCORES, rem)],
                            p_hbm.at[cid, pl.ds(stripe * NUM_SUBCORES, rem)])


def _tc_stats_body(p_ref, g_ref, dt_ref, c_ref, bias_ref, outr_ref, sums_ref):
    p0 = p_ref[0]
    p1 = p_ref[1]
    g = g_ref[:]
    a_src = g[:, 0:HEADS]
    xp = g[:, LANES:LANES + HEADS * HEAD_DIM]
    a_dst = dt_ref[:][:, 0:HEADS]
    wself = jnp.exp(_leaky(a_src + a_dst) - c_ref[:])
    den = p0[:, 0:HEADS] + p1[:, 0:HEADS] + wself

    col = lax.broadcasted_iota(jnp.int32, (HEADS, HEADS * HEAD_DIM), 1)
    hh = lax.broadcasted_iota(jnp.int32, (HEADS, HEAD_DIM * HEADS), 0)
    expand = jnp.where((col // HEAD_DIM) == hh, 1.0, 0.0)

    msg = (p0[:, LANES:] + p1[:, LANES:]
           + jnp.dot(wself, expand, preferred_element_type=jnp.float32) * xp)
    out = msg / (jnp.dot(den, expand, preferred_element_type=jnp.float32)
                 + 1e-16)
    out = out + bias_ref[:]
    outr_ref[:] = out
    sums_ref[0] = jnp.concatenate(
        [jnp.sum(out, axis=0, keepdims=True),
         jnp.sum(out * out, axis=0, keepdims=True)], axis=0)


def _tc_norm_body(n_rows, outr_ref, sums_ref, gamma_ref, beta_ref, out_ref):
    out = outr_ref[:]
    s = jnp.sum(sums_ref[:, 0, :], axis=0, keepdims=True)
    s2 = jnp.sum(sums_ref[:, 1, :], axis=0, keepdims=True)
    mean = s / n_rows
    var = s2 / n_rows - mean * mean
    out = (out - mean) * lax.rsqrt(var + 1e-5) * gamma_ref[:] + beta_ref[:]
    out_ref[:] = jnp.maximum(out, 0.0)


def kernel(x_gnn, edge_index, W, att_src, att_dst, bias, gamma, beta):
    n, in_ch = x_gnn.shape
    e = edge_index.shape[1]
    out_ch = W.shape[1]
    src = edge_index[0].astype(jnp.int32)
    dst = edge_index[1].astype(jnp.int32)

    g, dt, c = pl.pallas_call(
        _tc_pre_body,
        out_shape=[
            jax.ShapeDtypeStruct((n, ROW_W), jnp.float32),
            jax.ShapeDtypeStruct((n, LANES), jnp.float32),
            jax.ShapeDtypeStruct((n, HEADS), jnp.float32),
        ],
    )(x_gnn, W, att_src.reshape(1, out_ch), att_dst.reshape(1, out_ch))

    e_per_sub = e // (NUM_CORES * NUM_SUBCORES)
    nchunks = -(-e_per_sub // CHUNK) + 1
    nchunks = -(-nchunks // (2 * CPB)) * (2 * CPB)
    nrows = -(-e // CHUNK) + 2 * CPB
    pad = nrows * CHUNK - e
    src2 = jnp.concatenate([src, jnp.zeros((pad,), jnp.int32)]).reshape(
        nrows, CHUNK)
    dst2 = jnp.concatenate([dst, jnp.zeros((pad,), jnp.int32)]).reshape(
        nrows, CHUNK)
    zeros = jnp.zeros((n, ROW_W), jnp.float32)

    sc_call = pl.kernel(
        functools.partial(_sc_edge_body, n, e_per_sub, nchunks),
        out_type=jax.ShapeDtypeStruct((NUM_CORES, n, ROW_W), jnp.float32),
        mesh=plsc.VectorSubcoreMesh(core_axis_name="c", subcore_axis_name="s"),
        compiler_params=pltpu.CompilerParams(
            needs_layout_passes=False, use_tc_tiling_on_sc=False),
        scratch_types=(
            [pltpu.VMEM((CPB, CHUNK), jnp.int32)] * 4
            + [pltpu.VMEM((CHUNK, ROW_W), jnp.float32)] * NSLOTS
            + [pltpu.VMEM((CHUNK, LANES), jnp.float32)] * NSLOTS
            + [pltpu.VMEM((CHUNK, ROW_W), jnp.float32)] * NSLOTS
            + [pltpu.VMEM_SHARED((n, ROW_W), jnp.float32)]
            + [pltpu.SemaphoreType.DMA] * (3 * NSLOTS)
        ),
    )
    p = sc_call(g, dt, src2, dst2, zeros)

    blk = 1000
    nblk = n // blk
    outr, sums = pl.pallas_call(
        _tc_stats_body,
        grid=(nblk,),
        in_specs=[
            pl.BlockSpec((NUM_CORES, blk, ROW_W), lambda i: (0, i, 0)),
            pl.BlockSpec((blk, ROW_W), lambda i: (i, 0)),
            pl.BlockSpec((blk, LANES), lambda i: (i, 0)),
            pl.BlockSpec((blk, HEADS), lambda i: (i, 0)),
            pl.BlockSpec((1, out_ch), lambda i: (0, 0)),
        ],
        out_specs=[
            pl.BlockSpec((blk, out_ch), lambda i: (i, 0)),
            pl.BlockSpec((1, 2, out_ch), lambda i: (i, 0, 0)),
        ],
        out_shape=[
            jax.ShapeDtypeStruct((n, out_ch), jnp.float32),
            jax.ShapeDtypeStruct((nblk, 2, out_ch), jnp.float32),
        ],
    )(p, g, dt, c, bias.reshape(1, out_ch))

    out = pl.pallas_call(
        functools.partial(_tc_norm_body, float(n)),
        grid=(nblk,),
        in_specs=[
            pl.BlockSpec((blk, out_ch), lambda i: (i, 0)),
            pl.BlockSpec((nblk, 2, out_ch), lambda i: (0, 0, 0)),
            pl.BlockSpec((1, out_ch), lambda i: (0, 0)),
            pl.BlockSpec((1, out_ch), lambda i: (0, 0)),
        ],
        out_specs=pl.BlockSpec((blk, out_ch), lambda i: (i, 0)),
        out_shape=jax.ShapeDtypeStruct((n, out_ch), jnp.float32),
    )(outr, sums, gamma.reshape(1, out_ch), beta.reshape(1, out_ch))
    return out


# speedup vs baseline: 1.7612x; 1.0066x over previous
import functools

import jax
import jax.numpy as jnp
from jax import lax
from jax.experimental import pallas as pl
from jax.experimental.pallas import tpu as pltpu
from jax.experimental.pallas import tpu_sc as plsc

HEADS = 8
HEAD_DIM = 16
LANES = 16
NEG_SLOPE = 0.2

NUM_CORES = 2
NUM_SUBCORES = 16
CHUNK = 64
CPB = 8
NSLOTS = 2
ROW_W = HEADS * HEAD_DIM + LANES


def _leaky(x):
    return jnp.maximum(x, x * NEG_SLOPE)


def _tc_pre_body(x_ref, w_ref, asrc_ref, adst_ref, g_ref, dt_ref, c_ref):
    x = x_ref[:]
    xp = jnp.dot(x, w_ref[:], preferred_element_type=jnp.float32)
    ps = xp * asrc_ref[:]
    pd = xp * adst_ref[:]
    col = lax.broadcasted_iota(jnp.int32, (HEADS * HEAD_DIM, HEADS), 0)
    hh = lax.broadcasted_iota(jnp.int32, (HEADS * HEAD_DIM, HEADS), 1)
    ones_blk = jnp.where((col // HEAD_DIM) == hh, 1.0, 0.0)
    ones_rev = jnp.where((col // HEAD_DIM) == (HEADS - 1 - hh), 1.0, 0.0)
    a_src = jnp.dot(ps, ones_blk, preferred_element_type=jnp.float32)
    a_dst = jnp.dot(pd, ones_blk, preferred_element_type=jnp.float32)
    a_src_r = jnp.dot(ps, ones_rev, preferred_element_type=jnp.float32)
    a_dst_r = jnp.dot(pd, ones_rev, preferred_element_type=jnp.float32)
    amax = jnp.max(a_src, axis=0, keepdims=True)
    amax_r = jnp.max(a_src_r, axis=0, keepdims=True)
    c = _leaky(amax + a_dst)
    c_rev = _leaky(amax_r + a_dst_r)
    g_ref[:] = jnp.concatenate([a_src, jnp.zeros_like(a_src), xp], axis=1)
    dt_ref[:] = jnp.concatenate([a_dst, c_rev], axis=1)
    c_ref[:] = c


def _sc_edge_body(n_nodes, e_per_sub, nchunks,
                  g_hbm, dt_hbm, sd_hbm, p_hbm,
                  sdidx_a, sdidx_b, *bufs):
    gvs = bufs[0:NSLOTS]
    dbs = bufs[NSLOTS:2 * NSLOTS]
    mbs = bufs[2 * NSLOTS:3 * NSLOTS]
    acc = bufs[3 * NSLOTS]
    semg = bufs[3 * NSLOTS + 1:3 * NSLOTS + 1 + NSLOTS]
    semd = bufs[3 * NSLOTS + 1 + NSLOTS:3 * NSLOTS + 1 + 2 * NSLOTS]
    sems = bufs[3 * NSLOTS + 1 + 2 * NSLOTS:3 * NSLOTS + 1 + 3 * NSLOTS]
    cid = lax.axis_index("c")
    sid = lax.axis_index("s")

    zbuf = mbs[0]
    zvec = jnp.zeros((LANES,), jnp.float32)

    def zrow(k, carry):
        for cc in range(ROW_W // LANES):
            zbuf[k, pl.ds(cc * LANES, LANES)] = zvec
        return carry

    lax.fori_loop(0, CHUNK, zrow, 0, unroll=4)
    stripe = (n_nodes // (NUM_SUBCORES * 8)) * 8
    r_zero = sid * stripe
    rem = n_nodes - stripe * NUM_SUBCORES

    def zcopy(r_start, count):
        nfullz = count // CHUNK
        restz = count - nfullz * CHUNK

        def zc(q, carry):
            pltpu.sync_copy(zbuf, acc.at[pl.ds(r_start + q * CHUNK, CHUNK)])
            return carry

        lax.fori_loop(0, nfullz, zc, 0)
        if restz:
            pltpu.sync_copy(zbuf.at[pl.ds(0, restz)],
                            acc.at[pl.ds(r_start + nfullz * CHUNK, restz)])

    zcopy(r_zero, stripe)
    if rem:
        @pl.when(sid == NUM_SUBCORES - 1)
        def _():
            zcopy(stripe * NUM_SUBCORES, rem)
    plsc.subcore_barrier()

    a_lo = cid * (e_per_sub * NUM_SUBCORES) + sid * e_per_sub
    r0 = a_lo // CHUNK
    a64 = a_lo - r0 * CHUNK
    lane = lax.iota(jnp.int32, LANES)
    lo_mask = lane < HEADS

    idx_blks = (sdidx_a, sdidx_b)

    def copy_idx_block(first_chunk, blk):
        pltpu.sync_copy(sd_hbm.at[pl.ds(r0 + first_chunk, CPB)], blk)

    def issue_gather(blk, jr, s):
        pltpu.async_copy(g_hbm.at[blk.at[jr, 0]], gvs[s], semg[s])
        pltpu.async_copy(dt_hbm.at[blk.at[jr, 1]], dbs[s], semd[s])

    def wait_gather(s):
        pltpu.make_async_copy(g_hbm.at[pl.ds(0, CHUNK)], gvs[s],
                              semg[s]).wait()
        pltpu.make_async_copy(dt_hbm.at[pl.ds(0, CHUNK)], dbs[s],
                              semd[s]).wait()

    def wait_scatter(s):
        pltpu.make_async_copy(mbs[s], acc.at[pl.ds(0, CHUNK)],
                              sems[s]).wait()

    def compute_chunk(j, blk, jr, s):
        gv, db, mb = gvs[s], dbs[s], mbs[s]
        lo = a64 - j * CHUNK
        hi = lo + e_per_sub

        def edge_body(k, carry):
            s16 = gv[k, pl.ds(0, LANES)]
            d16 = db[k, :]
            v = s16 + d16
            t = _leaky(v) - lax.rev(v, (0,))
            w = jnp.where(lo_mask, jnp.exp(t), 0.0)
            w = w * ((k >= lo) & (k < hi)).astype(jnp.float32)
            mb[k, pl.ds(0, LANES)] = w
            xvs = [gv[k, pl.ds(LANES + h * LANES, LANES)]
                   for h in range(HEADS)]
            ws = [w[h] for h in range(HEADS)]
            for h in range(HEADS):
                mb[k, pl.ds(LANES + h * LANES, LANES)] = xvs[h] * ws[h]
            return carry

        lax.fori_loop(0, CHUNK, edge_body, 0, unroll=4)
        pltpu.async_copy(mb, acc.at[blk.at[jr, 1]], sems[s], add=True)

    copy_idx_block(0, idx_blks[0])
    for s in range(NSLOTS):
        issue_gather(idx_blks[0], s, s)

    nsuper = nchunks // (2 * CPB)

    def super_body(i, carry):
        for t in range(2 * CPB):
            j = i * (2 * CPB) + t
            s = t % NSLOTS
            blk = idx_blks[(t // CPB) % 2]
            wait_gather(s)

            @pl.when(j >= NSLOTS)
            def _():
                wait_scatter(s)

            compute_chunk(j, blk, t % CPB, s)

            tn = t + NSLOTS
            if tn == CPB:
                copy_idx_block(i * (2 * CPB) + CPB, idx_blks[1])
            if tn < 2 * CPB:
                issue_gather(idx_blks[(tn // CPB) % 2], tn % CPB, tn % NSLOTS)
            else:
                @pl.when(i < nsuper - 1)
                def _():
                    if tn == 2 * CPB:
                        copy_idx_block((i + 1) * (2 * CPB), idx_blks[0])
                    issue_gather(idx_blks[0], tn % CPB, tn % NSLOTS)
        return carry

    lax.fori_loop(0, nsuper, super_body, 0)
    for s in range(NSLOTS):
        wait_scatter(s)

    plsc.subcore_barrier()
    pltpu.sync_copy(acc.at[pl.ds(r_zero, stripe)],
                    p_hbm.at[cid, pl.ds(r_zero, stripe)])
    if rem:
        @pl.when(sid == NUM_SUBCORES - 1)
        def _():
            pltpu.sync_copy(acc.at[pl.ds(stripe * NUM_SUBCORES, rem)],
                            p_hbm.at[cid, pl.ds(stripe * NUM_SUBCORES, rem)])


def _tc_stats_body(p_ref, g_ref, dt_ref, c_ref, bias_ref, outr_ref, sums_ref):
    p0 = p_ref[0]
    p1 = p_ref[1]
    g = g_ref[:]
    a_src = g[:, 0:HEADS]
    xp = g[:, LANES:LANES + HEADS * HEAD_DIM]
    a_dst = dt_ref[:][:, 0:HEADS]
    wself = jnp.exp(_leaky(a_src + a_dst) - c_ref[:])
    den = p0[:, 0:HEADS] + p1[:, 0:HEADS] + wself

    col = lax.broadcasted_iota(jnp.int32, (HEADS, HEADS * HEAD_DIM), 1)
    hh = lax.broadcasted_iota(jnp.int32, (HEADS, HEAD_DIM * HEADS), 0)
    expand = jnp.where((col // HEAD_DIM) == hh, 1.0, 0.0)

    msg = (p0[:, LANES:] + p1[:, LANES:]
           + jnp.dot(wself, expand, preferred_element_type=jnp.float32) * xp)
    out = msg / (jnp.dot(den, expand, preferred_element_type=jnp.float32)
                 + 1e-16)
    out = out + bias_ref[:]
    outr_ref[:] = out
    sums_ref[0] = jnp.concatenate(
        [jnp.sum(out, axis=0, keepdims=True),
         jnp.sum(out * out, axis=0, keepdims=True)], axis=0)


def _tc_norm_body(n_rows, outr_ref, sums_ref, gamma_ref, beta_ref, out_ref):
    out = outr_ref[:]
    s = jnp.sum(sums_ref[:, 0, :], axis=0, keepdims=True)
    s2 = jnp.sum(sums_ref[:, 1, :], axis=0, keepdims=True)
    mean = s / n_rows
    var = s2 / n_rows - mean * mean
    out = (out - mean) * lax.rsqrt(var + 1e-5) * gamma_ref[:] + beta_ref[:]
    out_ref[:] = jnp.maximum(out, 0.0)


def kernel(x_gnn, edge_index, W, att_src, att_dst, bias, gamma, beta):
    n, in_ch = x_gnn.shape
    e = edge_index.shape[1]
    out_ch = W.shape[1]
    src = edge_index[0].astype(jnp.int32)
    dst = edge_index[1].astype(jnp.int32)

    g, dt, c = pl.pallas_call(
        _tc_pre_body,
        out_shape=[
            jax.ShapeDtypeStruct((n, ROW_W), jnp.float32),
            jax.ShapeDtypeStruct((n, LANES), jnp.float32),
            jax.ShapeDtypeStruct((n, HEADS), jnp.float32),
        ],
    )(x_gnn, W, att_src.reshape(1, out_ch), att_dst.reshape(1, out_ch))

    e_per_sub = e // (NUM_CORES * NUM_SUBCORES)
    nchunks = -(-e_per_sub // CHUNK) + 1
    nchunks = -(-nchunks // (2 * CPB)) * (2 * CPB)
    nrows = -(-e // CHUNK) + 2 * CPB
    pad = nrows * CHUNK - e
    src2 = jnp.concatenate([src, jnp.zeros((pad,), jnp.int32)]).reshape(
        nrows, 1, CHUNK)
    dst2 = jnp.concatenate([dst, jnp.zeros((pad,), jnp.int32)]).reshape(
        nrows, 1, CHUNK)
    sd2 = jnp.concatenate([src2, dst2], axis=1)

    sc_call = pl.kernel(
        functools.partial(_sc_edge_body, n, e_per_sub, nchunks),
        out_type=jax.ShapeDtypeStruct((NUM_CORES, n, ROW_W), jnp.float32),
        mesh=plsc.VectorSubcoreMesh(core_axis_name="c", subcore_axis_name="s"),
        compiler_params=pltpu.CompilerParams(
            needs_layout_passes=False, use_tc_tiling_on_sc=False),
        scratch_types=(
            [pltpu.VMEM((CPB, 2, CHUNK), jnp.int32)] * 2
            + [pltpu.VMEM((CHUNK, ROW_W), jnp.float32)] * NSLOTS
            + [pltpu.VMEM((CHUNK, LANES), jnp.float32)] * NSLOTS
            + [pltpu.VMEM((CHUNK, ROW_W), jnp.float32)] * NSLOTS
            + [pltpu.VMEM_SHARED((n, ROW_W), jnp.float32)]
            + [pltpu.SemaphoreType.DMA] * (3 * NSLOTS)
        ),
    )
    p = sc_call(g, dt, sd2)

    blk = 1000
    nblk = n // blk
    outr, sums = pl.pallas_call(
        _tc_stats_body,
        grid=(nblk,),
        in_specs=[
            pl.BlockSpec((NUM_CORES, blk, ROW_W), lambda i: (0, i, 0)),
            pl.BlockSpec((blk, ROW_W), lambda i: (i, 0)),
            pl.BlockSpec((blk, LANES), lambda i: (i, 0)),
            pl.BlockSpec((blk, HEADS), lambda i: (i, 0)),
            pl.BlockSpec((1, out_ch), lambda i: (0, 0)),
        ],
        out_specs=[
            pl.BlockSpec((blk, out_ch), lambda i: (i, 0)),
            pl.BlockSpec((1, 2, out_ch), lambda i: (i, 0, 0)),
        ],
        out_shape=[
            jax.ShapeDtypeStruct((n, out_ch), jnp.float32),
            jax.ShapeDtypeStruct((nblk, 2, out_ch), jnp.float32),
        ],
    )(p, g, dt, c, bias.reshape(1, out_ch))

    out = pl.pallas_call(
        functools.partial(_tc_norm_body, float(n)),
        grid=(nblk,),
        in_specs=[
            pl.BlockSpec((blk, out_ch), lambda i: (i, 0)),
            pl.BlockSpec((nblk, 2, out_ch), lambda i: (0, 0, 0)),
            pl.BlockSpec((1, out_ch), lambda i: (0, 0)),
            pl.BlockSpec((1, out_ch), lambda i: (0, 0)),
        ],
        out_specs=pl.BlockSpec((blk, out_ch), lambda i: (i, 0)),
        out_shape=jax.ShapeDtypeStruct((n, out_ch), jnp.float32),
    )(outr, sums, gamma.reshape(1, out_ch), beta.reshape(1, out_ch))
    return out
